# baseline (device time: 18522 ns/iter reference)
import jax
import jax.numpy as jnp
from jax import lax
from jax.experimental import pallas as pl
from jax.experimental.pallas import tpu as pltpu

N_DEV = 4
BLK = 64


def kernel(x, Wq, K_ext, V_ext, Wo):
    B, Sq_sh, Dm = x.shape
    _, Skv_sh, Hq, Dh = K_ext.shape
    HD = Hq * Dh

    K2 = K_ext.reshape(B, Skv_sh, HD)
    V2 = V_ext.reshape(B, Skv_sh, HD)

    def body(x_ref, wq_ref, k_ref, v_ref, wo_ref, out_ref,
             kbuf, vbuf, ks_sems, kr_sems, vs_sems, vr_sems):
        my = lax.axis_index("i")

        for b in range(B):
            kbuf[my, b] = k_ref[b].astype(jnp.float8_e4m3fn)
            vbuf[my, b] = v_ref[b].astype(jnp.bfloat16)

        barrier = pltpu.get_barrier_semaphore()
        for o in range(1, N_DEV):
            pl.semaphore_signal(
                barrier, inc=1,
                device_id=(lax.rem(my + o, N_DEV),),
                device_id_type=pl.DeviceIdType.MESH,
            )
        pl.semaphore_wait(barrier, N_DEV - 1)

        sends = []
        for j, o in enumerate(range(1, N_DEV)):
            peer = lax.rem(my + o, N_DEV)
            r = pltpu.make_async_remote_copy(
                src_ref=kbuf.at[my], dst_ref=kbuf.at[my],
                send_sem=ks_sems.at[j], recv_sem=kr_sems.at[j],
                device_id=(peer,), device_id_type=pl.DeviceIdType.MESH,
            )
            r.start()
            sends.append(r)
        for j, o in enumerate(range(1, N_DEV)):
            peer = lax.rem(my + o, N_DEV)
            r = pltpu.make_async_remote_copy(
                src_ref=vbuf.at[my], dst_ref=vbuf.at[my],
                send_sem=vs_sems.at[j], recv_sem=vr_sems.at[j],
                device_id=(peer,), device_id_type=pl.DeviceIdType.MESH,
            )
            r.start()
            sends.append(r)

        wq = wq_ref[...].astype(jnp.bfloat16)
        wo = wo_ref[...].astype(jnp.bfloat16)

        qbf = []
        for b in range(B):
            q_all = jnp.dot(
                x_ref[b].astype(jnp.bfloat16), wq,
                preferred_element_type=jnp.float32,
            ) * 0.125
            qbf.append(q_all.astype(jnp.bfloat16))

        qb = my * (Sq_sh // BLK) + \
            lax.broadcasted_iota(jnp.int32, (Sq_sh, Skv_sh), 0) // BLK
        jb = lax.broadcasted_iota(jnp.int32, (Sq_sh, Skv_sh), 1) // BLK

        ssum = [[jnp.zeros((Sq_sh, 1), jnp.float32)
                 for _ in range(Hq)] for _ in range(B)]
        ctx_acc = [[jnp.zeros((Sq_sh, Dh), jnp.float32)
                    for _ in range(Hq)] for _ in range(B)]

        def score_chunk(slot, kc_by_b):
            kbq = jb + slot * (Skv_sh // BLK)
            maskf = ((qb == kbq) | (kbq == 0) |
                     (lax.rem(qb + kbq, 3) == 0)).astype(jnp.float32)
            e_out = []
            for b in range(B):
                kc = kc_by_b[b]
                e_b = []
                for hh in range(Hq):
                    sc = lax.dot_general(
                        qbf[b][:, hh * Dh:(hh + 1) * Dh],
                        kc[:, hh * Dh:(hh + 1) * Dh],
                        (((1,), (1,)), ((), ())),
                        preferred_element_type=jnp.float32,
                    )
                    e = jnp.exp(sc) * maskf
                    ssum[b][hh] = ssum[b][hh] + jnp.sum(e, axis=1,
                                                        keepdims=True)
                    e_b.append(e.astype(jnp.bfloat16))
                e_out.append(e_b)
            return e_out

        def ctx_chunk(e_in, vc_by_b):
            for b in range(B):
                for hh in range(Hq):
                    ctx_acc[b][hh] = ctx_acc[b][hh] + jnp.dot(
                        e_in[b][hh], vc_by_b[b][:, hh * Dh:(hh + 1) * Dh],
                        preferred_element_type=jnp.float32,
                    )

        def wait(sems, j, buf, slot):
            r = pltpu.make_async_remote_copy(
                src_ref=buf.at[slot], dst_ref=buf.at[slot],
                send_sem=ks_sems.at[j],
                recv_sem=sems.at[j],
                device_id=(my,),
                device_id_type=pl.DeviceIdType.MESH,
            )
            r.wait_recv()

        e_local = score_chunk(my, [k_ref[b].astype(jnp.bfloat16)
                                   for b in range(B)])
        ctx_chunk(e_local, [v_ref[b].astype(jnp.bfloat16) for b in range(B)])

        ARRIVAL = ((0, 1), (2, 3), (1, 2))
        e_by_chunk = []
        for j, o in ARRIVAL:
            slot = lax.rem(my - o + N_DEV, N_DEV)
            wait(kr_sems, j, kbuf, slot)
            e_by_chunk.append(score_chunk(
                slot, [kbuf[slot, b].astype(jnp.bfloat16) for b in range(B)]))

        for c, (j, o) in enumerate(ARRIVAL):
            slot = lax.rem(my - o + N_DEV, N_DEV)
            wait(vr_sems, j, vbuf, slot)
            ctx_chunk(e_by_chunk[c], [vbuf[slot, b] for b in range(B)])

        for b in range(B):
            ctx = jnp.concatenate(
                [ctx_acc[b][hh] / ssum[b][hh] for hh in range(Hq)], axis=1,
            ).astype(jnp.bfloat16)
            out_ref[b] = jnp.dot(ctx, wo, preferred_element_type=jnp.float32)

        for r in sends:
            r.wait_send()

    return pl.pallas_call(
        body,
        out_shape=jax.ShapeDtypeStruct((B, Sq_sh, Dm), jnp.float32),
        in_specs=[pl.BlockSpec(memory_space=pltpu.VMEM)] * 5,
        out_specs=pl.BlockSpec(memory_space=pltpu.VMEM),
        scratch_shapes=[
            pltpu.VMEM((N_DEV, B, Skv_sh, HD), jnp.float8_e4m3fn),
            pltpu.VMEM((N_DEV, B, Skv_sh, HD), jnp.bfloat16),
            pltpu.SemaphoreType.DMA((N_DEV - 1,)),
            pltpu.SemaphoreType.DMA((N_DEV - 1,)),
            pltpu.SemaphoreType.DMA((N_DEV - 1,)),
            pltpu.SemaphoreType.DMA((N_DEV - 1,)),
        ],
        compiler_params=pltpu.CompilerParams(collective_id=0),
    )(x, Wq, K2, V2, Wo)


# device time: 14496 ns/iter; 1.2777x vs baseline; 1.2777x over previous
import jax
import jax.numpy as jnp
from jax import lax
from jax.experimental import pallas as pl
from jax.experimental.pallas import tpu as pltpu

N_DEV = 4
BLK = 64


def kernel(x, Wq, K_ext, V_ext, Wo):
    B, Sq_sh, Dm = x.shape
    _, Skv_sh, Hq, Dh = K_ext.shape
    HD = Hq * Dh

    K2 = K_ext.reshape(B, Skv_sh, HD)
    V2 = V_ext.reshape(B, Skv_sh, HD)

    def body(x_ref, wq_ref, k_ref, v_ref, wo_ref, out_ref,
             kbuf, vbuf, ks_sems, kr_sems, vs_sems, vr_sems):
        my = lax.axis_index("i")

        def quant(v):
            return jnp.clip(jnp.round(v * 32.0), -127.0, 127.0).astype(
                jnp.int8)

        for b in range(B):
            kbuf[my, b] = quant(k_ref[b])

        barrier = pltpu.get_barrier_semaphore()
        for o in range(1, N_DEV):
            pl.semaphore_signal(
                barrier, inc=1,
                device_id=(lax.rem(my + o, N_DEV),),
                device_id_type=pl.DeviceIdType.MESH,
            )
        pl.semaphore_wait(barrier, N_DEV - 1)

        sends = []
        for j, o in enumerate(range(1, N_DEV)):
            peer = lax.rem(my + o, N_DEV)
            r = pltpu.make_async_remote_copy(
                src_ref=kbuf.at[my], dst_ref=kbuf.at[my],
                send_sem=ks_sems.at[j], recv_sem=kr_sems.at[j],
                device_id=(peer,), device_id_type=pl.DeviceIdType.MESH,
            )
            r.start()
            sends.append(r)
        for b in range(B):
            vbuf[my, b] = quant(v_ref[b])
        for j, o in enumerate(range(1, N_DEV)):
            peer = lax.rem(my + o, N_DEV)
            r = pltpu.make_async_remote_copy(
                src_ref=vbuf.at[my], dst_ref=vbuf.at[my],
                send_sem=vs_sems.at[j], recv_sem=vr_sems.at[j],
                device_id=(peer,), device_id_type=pl.DeviceIdType.MESH,
            )
            r.start()
            sends.append(r)

        wq = wq_ref[...].astype(jnp.bfloat16)
        wo = wo_ref[...].astype(jnp.bfloat16)

        qbf = []
        for b in range(B):
            q_all = jnp.dot(
                x_ref[b].astype(jnp.bfloat16), wq,
                preferred_element_type=jnp.float32,
            ) * 0.125
            qbf.append(q_all.astype(jnp.bfloat16))

        qb = my * (Sq_sh // BLK) + \
            lax.broadcasted_iota(jnp.int32, (Sq_sh, Skv_sh), 0) // BLK
        jb = lax.broadcasted_iota(jnp.int32, (Sq_sh, Skv_sh), 1) // BLK

        ssum = [[jnp.zeros((Sq_sh, 1), jnp.float32)
                 for _ in range(Hq)] for _ in range(B)]
        ctx_acc = [[jnp.zeros((Sq_sh, Dh), jnp.float32)
                    for _ in range(Hq)] for _ in range(B)]

        def score_chunk(slot, kc_by_b):
            kbq = jb + slot * (Skv_sh // BLK)
            maskf = ((qb == kbq) | (kbq == 0) |
                     (lax.rem(qb + kbq, 3) == 0)).astype(jnp.float32)
            e_out = []
            for b in range(B):
                kc = kc_by_b[b]
                e_b = []
                for hh in range(Hq):
                    sc = lax.dot_general(
                        qbf[b][:, hh * Dh:(hh + 1) * Dh],
                        kc[:, hh * Dh:(hh + 1) * Dh],
                        (((1,), (1,)), ((), ())),
                        preferred_element_type=jnp.float32,
                    )
                    e = jnp.exp(sc) * maskf
                    ssum[b][hh] = ssum[b][hh] + jnp.sum(e, axis=1,
                                                        keepdims=True)
                    e_b.append(e.astype(jnp.bfloat16))
                e_out.append(e_b)
            return e_out

        def ctx_chunk(e_in, vc_by_b):
            for b in range(B):
                for hh in range(Hq):
                    ctx_acc[b][hh] = ctx_acc[b][hh] + jnp.dot(
                        e_in[b][hh], vc_by_b[b][:, hh * Dh:(hh + 1) * Dh],
                        preferred_element_type=jnp.float32,
                    )

        def wait(sems, j, buf, slot):
            r = pltpu.make_async_remote_copy(
                src_ref=buf.at[slot], dst_ref=buf.at[slot],
                send_sem=ks_sems.at[j],
                recv_sem=sems.at[j],
                device_id=(my,),
                device_id_type=pl.DeviceIdType.MESH,
            )
            r.wait_recv()

        e_local = score_chunk(my, [k_ref[b].astype(jnp.bfloat16)
                                   for b in range(B)])
        ctx_chunk(e_local, [v_ref[b].astype(jnp.bfloat16) for b in range(B)])

        ARRIVAL = ((0, 1), (2, 3), (1, 2))
        e_by_chunk = []
        for j, o in ARRIVAL:
            slot = lax.rem(my - o + N_DEV, N_DEV)
            wait(kr_sems, j, kbuf, slot)
            e_by_chunk.append(score_chunk(
                slot, [kbuf[slot, b].astype(jnp.bfloat16) * (1.0 / 32.0)
                       for b in range(B)]))

        for c, (j, o) in enumerate(ARRIVAL):
            slot = lax.rem(my - o + N_DEV, N_DEV)
            wait(vr_sems, j, vbuf, slot)
            ctx_chunk(e_by_chunk[c],
                      [vbuf[slot, b].astype(jnp.bfloat16) * (1.0 / 32.0)
                       for b in range(B)])

        for b in range(B):
            ctx = jnp.concatenate(
                [ctx_acc[b][hh] / ssum[b][hh] for hh in range(Hq)], axis=1,
            ).astype(jnp.bfloat16)
            out_ref[b] = jnp.dot(ctx, wo, preferred_element_type=jnp.float32)

        for r in sends:
            r.wait_send()

    return pl.pallas_call(
        body,
        out_shape=jax.ShapeDtypeStruct((B, Sq_sh, Dm), jnp.float32),
        in_specs=[pl.BlockSpec(memory_space=pltpu.VMEM)] * 5,
        out_specs=pl.BlockSpec(memory_space=pltpu.VMEM),
        scratch_shapes=[
            pltpu.VMEM((N_DEV, B, Skv_sh, HD), jnp.int8),
            pltpu.VMEM((N_DEV, B, Skv_sh, HD), jnp.int8),
            pltpu.SemaphoreType.DMA((N_DEV - 1,)),
            pltpu.SemaphoreType.DMA((N_DEV - 1,)),
            pltpu.SemaphoreType.DMA((N_DEV - 1,)),
            pltpu.SemaphoreType.DMA((N_DEV - 1,)),
        ],
        compiler_params=pltpu.CompilerParams(collective_id=0),
    )(x, Wq, K2, V2, Wo)
